# Initial kernel scaffold; baseline (speedup 1.0000x reference)
import functools

import jax
import jax.numpy as jnp
from jax import lax
from jax.experimental import pallas as pl
from jax.experimental.pallas import tpu as pltpu
from jax.experimental.pallas import tpu_sc as plsc

_B, _N, _M = 4, 4096, 4096
_C = 64
_CH = 128
_NG = _N // _CH
_G = _B * _NG
_R = _B * _N
_RB = 1024
_EPS = 1e-5


def _top3_kernel(q_ref, p_ref, idx_ref, wn_ref):
    b = pl.program_id(0)
    q = q_ref[0]
    p = p_ref[0]
    dx = q[:, 0:1] - p[0:1, :]
    dy = q[:, 1:2] - p[1:2, :]
    dz = q[:, 2:3] - p[2:3, :]
    d2 = (dx * dx + dy * dy) + dz * dz
    d = jnp.sqrt(d2)
    col = lax.broadcasted_iota(jnp.int32, d.shape, 1)
    inf = jnp.float32(jnp.inf)
    m_i32 = jnp.int32(_M)
    dwork = d
    dks = []
    iks = []
    for k in range(3):
        dk = jnp.min(dwork, axis=1, keepdims=True)
        ik = jnp.min(jnp.where(dwork == dk, col, m_i32), axis=1,
                     keepdims=True)
        dks.append(dk)
        iks.append(ik)
        if k < 2:
            dwork = jnp.where(col == ik, inf, dwork)
    w0 = 1.0 / dks[0]
    w1 = 1.0 / dks[1]
    w2 = 1.0 / dks[2]
    wsum = w0 + w1 + w2
    is_big = dks[0] == 0.0
    wn0 = jnp.where(is_big, 1.0, w0 / wsum)
    wn1 = jnp.where(is_big, 0.0, w1 / wsum)
    wn2 = jnp.where(is_big, 0.0, w2 / wsum)
    base = b * m_i32
    zi = jnp.zeros_like(iks[0])
    zf = jnp.zeros_like(wn0)
    idx_ref[0] = jnp.concatenate(
        [iks[0] + base, iks[1] + base, iks[2] + base, zi, zi, zi, zi, zi],
        axis=1)
    wn_ref[0] = jnp.concatenate([wn0, wn1, wn2, zf, zf, zf, zf, zf], axis=1)


def _top3_call(q_pad, p_pad):
    return pl.pallas_call(
        _top3_kernel,
        grid=(_B, _N // _CH),
        in_specs=[
            pl.BlockSpec((1, _CH, 8), lambda b, i: (b, i, 0)),
            pl.BlockSpec((1, 8, _M), lambda b, i: (b, 0, 0)),
        ],
        out_specs=[
            pl.BlockSpec((1, _CH, 8), lambda b, i: (b, i, 0)),
            pl.BlockSpec((1, _CH, 8), lambda b, i: (b, i, 0)),
        ],
        out_shape=[
            jax.ShapeDtypeStruct((_B, _N, 8), jnp.int32),
            jax.ShapeDtypeStruct((_B, _N, 8), jnp.float32),
        ],
    )(q_pad, p_pad)


def _interp_body(fea_hbm, idx_hbm, wn_hbm, out_hbm, idx_v, wn_v, rows_v,
                 out_v, sem):
    info = plsc.get_sparse_core_info()
    nw = info.num_cores * info.num_subcores
    wid = lax.axis_index("s") * info.num_cores + lax.axis_index("c")
    gpw = _G // nw
    for gi in range(gpw):
        g = wid * gpw + gi
        pltpu.sync_copy(idx_hbm.at[g], idx_v)
        pltpu.sync_copy(wn_hbm.at[g], wn_v)
        for k in range(3):
            pltpu.async_copy(fea_hbm.at[idx_v.at[k]], rows_v.at[k],
                             sem).wait()

        def body(qq, carry):
            w0 = wn_v[0, qq]
            w1 = wn_v[1, qq]
            w2 = wn_v[2, qq]
            for cc in range(_C // 16):
                sl = pl.ds(cc * 16, 16)
                acc = w0 * rows_v[0, qq, sl]
                acc = acc + w1 * rows_v[1, qq, sl]
                acc = acc + w2 * rows_v[2, qq, sl]
                out_v[qq, sl] = acc
            return carry

        lax.fori_loop(0, _CH, body, 0)
        pltpu.sync_copy(out_v, out_hbm.at[pl.ds(g * _CH, _CH)])


def _interp_call(fea_flat, idx3, wn3):
    mesh = plsc.VectorSubcoreMesh(core_axis_name="c", subcore_axis_name="s")
    kern = pl.kernel(
        _interp_body,
        out_type=jax.ShapeDtypeStruct((_R, _C), jnp.float32),
        mesh=mesh,
        scratch_types=[
            pltpu.VMEM((3, _CH), jnp.int32),
            pltpu.VMEM((3, _CH), jnp.float32),
            pltpu.VMEM((3, _CH, _C), jnp.float32),
            pltpu.VMEM((_CH, _C), jnp.float32),
            pltpu.SemaphoreType.DMA,
        ],
    )
    return kern(fea_flat, idx3, wn3)


def _mlp0_kernel(xs_ref, xf_ref, w_ref, b_ref, h_ref, st_ref):
    i = pl.program_id(0)
    x = jnp.concatenate([xs_ref[...], xf_ref[...]], axis=1)
    h = lax.dot_general(x, w_ref[...], (((1,), (1,)), ((), ())),
                        preferred_element_type=jnp.float32) + b_ref[...]
    h_ref[...] = h

    @pl.when(i == 0)
    def _():
        st_ref[...] = jnp.zeros_like(st_ref)

    st_ref[0:1, :] += jnp.sum(h, axis=0, keepdims=True)
    st_ref[1:2, :] += jnp.sum(h * h, axis=0, keepdims=True)


def _mlp1_kernel(h0_ref, st0_ref, g_ref, be_ref, w_ref, b_ref, h1_ref,
                 st1_ref):
    i = pl.program_id(0)
    rn = jnp.float32(_R)
    mean = st0_ref[0:1, :] / rn
    var = st0_ref[1:2, :] / rn - mean * mean
    scale = g_ref[...] / jnp.sqrt(var + _EPS)
    xn = (h0_ref[...] - mean) * scale + be_ref[...]
    xn = jnp.maximum(xn, 0.0)
    h = lax.dot_general(xn, w_ref[...], (((1,), (1,)), ((), ())),
                        preferred_element_type=jnp.float32) + b_ref[...]
    h1_ref[...] = h

    @pl.when(i == 0)
    def _():
        st1_ref[...] = jnp.zeros_like(st1_ref)

    st1_ref[0:1, :] += jnp.sum(h, axis=0, keepdims=True)
    st1_ref[1:2, :] += jnp.sum(h * h, axis=0, keepdims=True)


def _bn_out_kernel(h1_ref, st1_ref, g_ref, be_ref, o_ref):
    rn = jnp.float32(_R)
    mean = st1_ref[0:1, :] / rn
    var = st1_ref[1:2, :] / rn - mean * mean
    scale = g_ref[...] / jnp.sqrt(var + _EPS)
    xn = (h1_ref[...] - mean) * scale + be_ref[...]
    o_ref[...] = jnp.maximum(xn, 0.0)


def _mlp_call(xs, xf, W0, b0, gamma0, beta0, W1, b1, gamma1, beta1):
    grid = (_R // _RB,)
    row_spec = pl.BlockSpec((_RB, _C), lambda i: (i, 0))
    row_spec128 = pl.BlockSpec((_RB, 128), lambda i: (i, 0))
    full_spec = pl.BlockSpec((128, 128), lambda i: (0, 0))
    vec_spec = pl.BlockSpec((1, 128), lambda i: (0, 0))
    st_spec = pl.BlockSpec((8, 128), lambda i: (0, 0))

    h0, st0 = pl.pallas_call(
        _mlp0_kernel,
        grid=grid,
        in_specs=[row_spec, row_spec, full_spec, vec_spec],
        out_specs=[row_spec128, st_spec],
        out_shape=[
            jax.ShapeDtypeStruct((_R, 128), jnp.float32),
            jax.ShapeDtypeStruct((8, 128), jnp.float32),
        ],
    )(xs, xf, W0, b0)

    h1, st1 = pl.pallas_call(
        _mlp1_kernel,
        grid=grid,
        in_specs=[row_spec128, st_spec, vec_spec, vec_spec, full_spec,
                  vec_spec],
        out_specs=[row_spec128, st_spec],
        out_shape=[
            jax.ShapeDtypeStruct((_R, 128), jnp.float32),
            jax.ShapeDtypeStruct((8, 128), jnp.float32),
        ],
    )(h0, st0, gamma0, beta0, W1, b1)

    out = pl.pallas_call(
        _bn_out_kernel,
        grid=grid,
        in_specs=[row_spec128, st_spec, vec_spec, vec_spec],
        out_specs=row_spec128,
        out_shape=jax.ShapeDtypeStruct((_R, 128), jnp.float32),
    )(h1, st1, gamma1, beta1)
    return out


def kernel(pts_co_small, pts_fea_small, pts_co_big, pts_fea_big,
           W0, b0, gamma0, beta0, W1, b1, gamma1, beta1):
    q_pad = jnp.pad(pts_co_small, ((0, 0), (0, 0), (0, 5)))
    p_pad = jnp.pad(jnp.transpose(pts_co_big, (0, 2, 1)),
                    ((0, 0), (0, 5), (0, 0)))

    idx8, wn8 = _top3_call(q_pad, p_pad)

    idx3 = jnp.transpose(idx8.reshape(_G, _CH, 8)[:, :, :3], (0, 2, 1))
    wn3 = jnp.transpose(wn8.reshape(_G, _CH, 8)[:, :, :3], (0, 2, 1))
    fea_flat = pts_fea_big.reshape(_B * _M, _C)

    feats = _interp_call(fea_flat, idx3, wn3)

    xs = pts_fea_small.reshape(_R, _C)
    out = _mlp_call(xs, feats, W0, b0.reshape(1, 128),
                    gamma0.reshape(1, 128), beta0.reshape(1, 128),
                    W1, b1.reshape(1, 128), gamma1.reshape(1, 128),
                    beta1.reshape(1, 128))
    return out.reshape(_B, _N, 128)


# speedup vs baseline: 45.3975x; 45.3975x over previous
import functools

import jax
import jax.numpy as jnp
from jax import lax
from jax.experimental import pallas as pl
from jax.experimental.pallas import tpu as pltpu
from jax.experimental.pallas import tpu_sc as plsc

_B, _N, _M = 4, 4096, 4096
_C = 64
_CH = 128
_NG = _N // _CH
_G = _B * _NG
_R = _B * _N
_RB = 1024
_EPS = 1e-5


def _top3_kernel(q_ref, p_ref, idx_ref, wn_ref):
    b = pl.program_id(0)
    q = q_ref[0]
    p = p_ref[0]
    dx = q[:, 0:1] - p[0:1, :]
    dy = q[:, 1:2] - p[1:2, :]
    dz = q[:, 2:3] - p[2:3, :]
    d2 = (dx * dx + dy * dy) + dz * dz
    d = jnp.sqrt(d2)
    col = lax.broadcasted_iota(jnp.int32, d.shape, 1)
    inf = jnp.float32(jnp.inf)
    m_i32 = jnp.int32(_M)
    dwork = d
    dks = []
    iks = []
    for k in range(3):
        dk = jnp.min(dwork, axis=1, keepdims=True)
        ik = jnp.min(jnp.where(dwork == dk, col, m_i32), axis=1,
                     keepdims=True)
        dks.append(dk)
        iks.append(ik)
        if k < 2:
            dwork = jnp.where(col == ik, inf, dwork)
    w0 = 1.0 / dks[0]
    w1 = 1.0 / dks[1]
    w2 = 1.0 / dks[2]
    wsum = w0 + w1 + w2
    is_big = dks[0] == 0.0
    wn0 = jnp.where(is_big, 1.0, w0 / wsum)
    wn1 = jnp.where(is_big, 0.0, w1 / wsum)
    wn2 = jnp.where(is_big, 0.0, w2 / wsum)
    base = b * m_i32
    zi = jnp.zeros_like(iks[0])
    zf = jnp.zeros_like(wn0)
    idx_ref[0] = jnp.concatenate(
        [iks[0] + base, iks[1] + base, iks[2] + base, zi, zi, zi, zi, zi],
        axis=1)
    wn_ref[0] = jnp.concatenate([wn0, wn1, wn2, zf, zf, zf, zf, zf], axis=1)


def _top3_call(q_pad, p_pad):
    return pl.pallas_call(
        _top3_kernel,
        grid=(_B, _N // _CH),
        in_specs=[
            pl.BlockSpec((1, _CH, 8), lambda b, i: (b, i, 0)),
            pl.BlockSpec((1, 8, _M), lambda b, i: (b, 0, 0)),
        ],
        out_specs=[
            pl.BlockSpec((1, _CH, 8), lambda b, i: (b, i, 0)),
            pl.BlockSpec((1, _CH, 8), lambda b, i: (b, i, 0)),
        ],
        out_shape=[
            jax.ShapeDtypeStruct((_B, _N, 8), jnp.int32),
            jax.ShapeDtypeStruct((_B, _N, 8), jnp.float32),
        ],
    )(q_pad, p_pad)


def _interp_body(fea_hbm, idx_hbm, wn_hbm, out_hbm, idx_v, wn_v, rows_v,
                 out_v, sem):
    info = plsc.get_sparse_core_info()
    nw = info.num_cores * info.num_subcores
    wid = lax.axis_index("s") * info.num_cores + lax.axis_index("c")
    gpw = _G // nw
    for gi in range(gpw):
        g = wid * gpw + gi
        pltpu.sync_copy(idx_hbm.at[g], idx_v)
        pltpu.sync_copy(wn_hbm.at[g], wn_v)
        for k in range(3):
            pltpu.async_copy(fea_hbm.at[idx_v.at[k]], rows_v.at[k],
                             sem).wait()

        def body(t, carry):
            q0 = t * 16
            wv0 = wn_v[0, pl.ds(q0, 16)]
            wv1 = wn_v[1, pl.ds(q0, 16)]
            wv2 = wn_v[2, pl.ds(q0, 16)]
            for j in range(16):
                qq = q0 + j
                w0 = wv0[j]
                w1 = wv1[j]
                w2 = wv2[j]
                for cc in range(_C // 16):
                    sl = pl.ds(cc * 16, 16)
                    acc = w0 * rows_v[0, qq, sl]
                    acc = acc + w1 * rows_v[1, qq, sl]
                    acc = acc + w2 * rows_v[2, qq, sl]
                    out_v[qq, sl] = acc
            return carry

        lax.fori_loop(0, _CH // 16, body, 0)
        pltpu.sync_copy(out_v, out_hbm.at[pl.ds(g * _CH, _CH)])


def _interp_call(fea_flat, idx3, wn3):
    mesh = plsc.VectorSubcoreMesh(core_axis_name="c", subcore_axis_name="s")
    fea_flat = jnp.pad(fea_flat, ((0, 0), (0, 128 - _C)))
    kern = pl.kernel(
        _interp_body,
        out_type=jax.ShapeDtypeStruct((_R, _C), jnp.float32),
        mesh=mesh,
        scratch_types=[
            pltpu.VMEM((3, _CH), jnp.int32),
            pltpu.VMEM((3, _CH), jnp.float32),
            pltpu.VMEM((3, _CH, 128), jnp.float32),
            pltpu.VMEM((_CH, _C), jnp.float32),
            pltpu.SemaphoreType.DMA,
        ],
    )
    return kern(fea_flat, idx3, wn3)


def _mlp0_kernel(xs_ref, xf_ref, w_ref, b_ref, h_ref, st_ref):
    i = pl.program_id(0)
    x = jnp.concatenate([xs_ref[...], xf_ref[...]], axis=1)
    h = lax.dot_general(x, w_ref[...], (((1,), (1,)), ((), ())),
                        preferred_element_type=jnp.float32) + b_ref[...]
    h_ref[...] = h

    @pl.when(i == 0)
    def _():
        st_ref[...] = jnp.zeros_like(st_ref)

    st_ref[0:1, :] += jnp.sum(h, axis=0, keepdims=True)
    st_ref[1:2, :] += jnp.sum(h * h, axis=0, keepdims=True)


def _mlp1_kernel(h0_ref, st0_ref, g_ref, be_ref, w_ref, b_ref, h1_ref,
                 st1_ref):
    i = pl.program_id(0)
    rn = jnp.float32(_R)
    mean = st0_ref[0:1, :] / rn
    var = st0_ref[1:2, :] / rn - mean * mean
    scale = g_ref[...] / jnp.sqrt(var + _EPS)
    xn = (h0_ref[...] - mean) * scale + be_ref[...]
    xn = jnp.maximum(xn, 0.0)
    h = lax.dot_general(xn, w_ref[...], (((1,), (1,)), ((), ())),
                        preferred_element_type=jnp.float32) + b_ref[...]
    h1_ref[...] = h

    @pl.when(i == 0)
    def _():
        st1_ref[...] = jnp.zeros_like(st1_ref)

    st1_ref[0:1, :] += jnp.sum(h, axis=0, keepdims=True)
    st1_ref[1:2, :] += jnp.sum(h * h, axis=0, keepdims=True)


def _bn_out_kernel(h1_ref, st1_ref, g_ref, be_ref, o_ref):
    rn = jnp.float32(_R)
    mean = st1_ref[0:1, :] / rn
    var = st1_ref[1:2, :] / rn - mean * mean
    scale = g_ref[...] / jnp.sqrt(var + _EPS)
    xn = (h1_ref[...] - mean) * scale + be_ref[...]
    o_ref[...] = jnp.maximum(xn, 0.0)


def _mlp_call(xs, xf, W0, b0, gamma0, beta0, W1, b1, gamma1, beta1):
    grid = (_R // _RB,)
    row_spec = pl.BlockSpec((_RB, _C), lambda i: (i, 0))
    row_spec128 = pl.BlockSpec((_RB, 128), lambda i: (i, 0))
    full_spec = pl.BlockSpec((128, 128), lambda i: (0, 0))
    vec_spec = pl.BlockSpec((1, 128), lambda i: (0, 0))
    st_spec = pl.BlockSpec((8, 128), lambda i: (0, 0))

    h0, st0 = pl.pallas_call(
        _mlp0_kernel,
        grid=grid,
        in_specs=[row_spec, row_spec, full_spec, vec_spec],
        out_specs=[row_spec128, st_spec],
        out_shape=[
            jax.ShapeDtypeStruct((_R, 128), jnp.float32),
            jax.ShapeDtypeStruct((8, 128), jnp.float32),
        ],
    )(xs, xf, W0, b0)

    h1, st1 = pl.pallas_call(
        _mlp1_kernel,
        grid=grid,
        in_specs=[row_spec128, st_spec, vec_spec, vec_spec, full_spec,
                  vec_spec],
        out_specs=[row_spec128, st_spec],
        out_shape=[
            jax.ShapeDtypeStruct((_R, 128), jnp.float32),
            jax.ShapeDtypeStruct((8, 128), jnp.float32),
        ],
    )(h0, st0, gamma0, beta0, W1, b1)

    out = pl.pallas_call(
        _bn_out_kernel,
        grid=grid,
        in_specs=[row_spec128, st_spec, vec_spec, vec_spec],
        out_specs=row_spec128,
        out_shape=jax.ShapeDtypeStruct((_R, 128), jnp.float32),
    )(h1, st1, gamma1, beta1)
    return out


def kernel(pts_co_small, pts_fea_small, pts_co_big, pts_fea_big,
           W0, b0, gamma0, beta0, W1, b1, gamma1, beta1):
    q_pad = jnp.pad(pts_co_small, ((0, 0), (0, 0), (0, 5)))
    p_pad = jnp.pad(jnp.transpose(pts_co_big, (0, 2, 1)),
                    ((0, 0), (0, 5), (0, 0)))

    idx8, wn8 = _top3_call(q_pad, p_pad)

    idx3 = jnp.transpose(idx8.reshape(_G, _CH, 8)[:, :, :3], (0, 2, 1))
    wn3 = jnp.transpose(wn8.reshape(_G, _CH, 8)[:, :, :3], (0, 2, 1))
    fea_flat = pts_fea_big.reshape(_B * _M, _C)

    feats = _interp_call(fea_flat, idx3, wn3)

    xs = pts_fea_small.reshape(_R, _C)
    out = _mlp_call(xs, feats, W0, b0.reshape(1, 128),
                    gamma0.reshape(1, 128), beta0.reshape(1, 128),
                    W1, b1.reshape(1, 128), gamma1.reshape(1, 128),
                    beta1.reshape(1, 128))
    return out.reshape(_B, _N, 128)
